# Initial kernel scaffold; baseline (speedup 1.0000x reference)
import jax
import jax.numpy as jnp
from jax.experimental import pallas as pl


def kernel(h, W, b):
    raise NotImplementedError("write your pallas kernel here")


# speedup vs baseline: 1.4813x; 1.4813x over previous
import functools

import jax
import jax.numpy as jnp
from jax import lax
from jax.experimental import pallas as pl

B, N, C = 16, 4096, 512
K = N // 2
T = 512
IT = N // T
KT = K // T


def _scores_body(h_ref, w_ref, b_ref, s_ref):
    hmat = h_ref[0]
    wt = w_ref[...]
    logits = lax.dot_general(
        hmat, wt, (((1,), (0,)), ((), ())),
        preferred_element_type=jnp.float32)
    s = jax.nn.sigmoid(logits + b_ref[0])
    s_ref[...] = s.reshape(1, 1, N)


def _rank_body(si_ref, sj_ref, rank_ref):
    it = pl.program_id(1)
    jt = pl.program_id(2)

    @pl.when(jt == 0)
    def _():
        rank_ref[...] = jnp.zeros_like(rank_ref)

    s_i = si_ref[...].reshape(T, 1)
    s_j = sj_ref[...].reshape(1, T)
    ig = it * T + lax.broadcasted_iota(jnp.int32, (T, 1), 0)
    jg = jt * T + lax.broadcasted_iota(jnp.int32, (1, T), 1)
    gt = (s_j > s_i).astype(jnp.int32)
    tie = ((s_j == s_i) & (jg < ig)).astype(jnp.int32)
    part = jnp.sum(gt + tie, axis=1, keepdims=True)
    rank_ref[...] += part.reshape(1, 1, T)


def _gather_body(rank_ref, s_ref, h_ref, out_ref):
    jt = pl.program_id(1)

    @pl.when(jt == 0)
    def _():
        out_ref[...] = jnp.zeros_like(out_ref)

    rank_row = rank_ref[...].reshape(1, T)
    s_row = s_ref[...].reshape(1, T)
    hmat = h_ref[0]
    for kt in range(KT):
        kio = kt * T + lax.broadcasted_iota(jnp.int32, (T, 1), 0)
        pmat = jnp.where(rank_row == kio, s_row, 0.0)
        out_ref[0, kt * T:(kt + 1) * T, :] += lax.dot_general(
            pmat, hmat, (((1,), (0,)), ((), ())),
            preferred_element_type=jnp.float32)


@jax.jit
def kernel(h, W, b):
    wt = W.reshape(C, 1)
    scores = pl.pallas_call(
        _scores_body,
        grid=(B,),
        in_specs=[
            pl.BlockSpec((1, N, C), lambda i: (i, 0, 0)),
            pl.BlockSpec((C, 1), lambda i: (0, 0)),
            pl.BlockSpec((1,), lambda i: (0,)),
        ],
        out_specs=pl.BlockSpec((1, 1, N), lambda i: (i, 0, 0)),
        out_shape=jax.ShapeDtypeStruct((B, 1, N), jnp.float32),
    )(h, wt, b)

    ranks = pl.pallas_call(
        _rank_body,
        grid=(B, IT, IT),
        in_specs=[
            pl.BlockSpec((1, 1, T), lambda b_, i, j: (b_, 0, i)),
            pl.BlockSpec((1, 1, T), lambda b_, i, j: (b_, 0, j)),
        ],
        out_specs=pl.BlockSpec((1, 1, T), lambda b_, i, j: (b_, 0, i)),
        out_shape=jax.ShapeDtypeStruct((B, 1, N), jnp.int32),
    )(scores, scores)

    out = pl.pallas_call(
        _gather_body,
        grid=(B, IT),
        in_specs=[
            pl.BlockSpec((1, 1, T), lambda b_, j: (b_, 0, j)),
            pl.BlockSpec((1, 1, T), lambda b_, j: (b_, 0, j)),
            pl.BlockSpec((1, T, C), lambda b_, j: (b_, j, 0)),
        ],
        out_specs=pl.BlockSpec((1, K, C), lambda b_, j: (b_, 0, 0)),
        out_shape=jax.ShapeDtypeStruct((B, K, C), jnp.float32),
    )(ranks, scores, h)
    return out
